# Initial kernel scaffold; baseline (speedup 1.0000x reference)
import jax
import jax.numpy as jnp
from jax.experimental import pallas as pl


def kernel(nfeats, efeats, edge_index, Wm1, bm1, Wa1, ba1, Wm2, bm2, Wa2, ba2, Wm3, bm3, Wa3, ba3):
    raise NotImplementedError("write your pallas kernel here")


# speedup vs baseline: 4.3100x; 4.3100x over previous
import functools

import jax
import jax.numpy as jnp
from jax import lax
from jax.experimental import pallas as pl
from jax.experimental.pallas import tpu as pltpu
from jax.experimental.pallas import tpu_sc as plsc

N = 10000
NP = 10240
E = 320000
DIN = 128
DE = 16

NC = 2
NS = 16
NW = NC * NS
EPW = E // NW
K = 40
NCH = EPW // K
RPT = NP // NS
ZR = 128


@functools.cache
def _edge_kernel(dp: int):
    mesh = plsc.VectorSubcoreMesh(core_axis_name="c", subcore_axis_name="s")

    @functools.partial(
        pl.kernel,
        out_type=jax.ShapeDtypeStruct((NC, NP, dp), jnp.float32),
        mesh=mesh,
        compiler_params=pltpu.CompilerParams(use_tc_tiling_on_sc=False),
        scratch_types=[
            pltpu.VMEM_SHARED((NP, dp), jnp.float32),
            pltpu.VMEM((ZR, dp), jnp.float32),
            pltpu.VMEM((K,), jnp.int32),
            pltpu.VMEM((K,), jnp.int32),
            pltpu.VMEM((K,), jnp.int32),
            pltpu.VMEM((K,), jnp.int32),
            pltpu.VMEM((K, dp), jnp.float32),
            pltpu.VMEM((K, dp), jnp.float32),
            pltpu.SemaphoreType.DMA,
            pltpu.SemaphoreType.DMA,
        ],
    )
    def body(p_hbm, q_hbm, src_hbm, dst_hbm, out_hbm,
             acc, zb, s0, s1, d0, d1, m0, m1, sem0, sem1):
        c = lax.axis_index("c")
        s = lax.axis_index("s")
        w = c * NS + s
        estart = w * EPW
        nj = dp // 16
        zv = jnp.zeros((16,), jnp.float32)

        def zrow(r, _):
            for j in range(nj):
                zb[r, pl.ds(j * 16, 16)] = zv
            return 0
        lax.fori_loop(0, ZR, zrow, 0)
        rbase = s * RPT
        for z in range(RPT // ZR):
            pltpu.sync_copy(zb, acc.at[pl.ds(rbase + z * ZR, ZR)])
        plsc.subcore_barrier()

        bufs = ((s0, d0, m0, sem0), (s1, d1, m1, sem1))

        def stage(i, b):
            sb, db, mb, sem = bufs[b]
            off = estart + i * K
            pltpu.async_copy(src_hbm.at[pl.ds(off, K)], sb, sem)
            pltpu.async_copy(dst_hbm.at[pl.ds(off, K)], db, sem)
            pltpu.async_copy(q_hbm.at[pl.ds(off, K)], mb, sem)

        def process(b):
            sb, db, mb, sem = bufs[b]
            pltpu.make_async_copy(src_hbm.at[pl.ds(0, K)], sb, sem).wait()
            pltpu.make_async_copy(dst_hbm.at[pl.ds(0, K)], db, sem).wait()
            pltpu.make_async_copy(q_hbm.at[pl.ds(0, K)], mb, sem).wait()
            pltpu.async_copy(p_hbm.at[sb], mb, sem, add=True).wait()
            def rrow(r4, _):
                for u in range(4):
                    for j in range(nj):
                        sl = pl.ds(j * 16, 16)
                        v = mb[r4 * 4 + u, sl]
                        mb[r4 * 4 + u, sl] = jnp.maximum(v, zv)
                return 0
            lax.fori_loop(0, K // 4, rrow, 0)
            pltpu.sync_copy(mb, acc.at[db], add=True)

        stage(0, 0)

        def piped(g, _):
            i = 2 * g
            stage(i + 1, 1)
            process(0)
            stage(i + 2, 0)
            process(1)
            return 0
        lax.fori_loop(0, NCH // 2 - 1, piped, 0)
        stage(NCH - 1, 1)
        process(0)
        process(1)

        plsc.subcore_barrier()
        pltpu.sync_copy(acc.at[pl.ds(rbase, RPT)],
                        out_hbm.at[c, pl.ds(rbase, RPT)])

    return body


def _q_prep(ef, w1, b1, w2, b2, w3, b3):
    BR = 8000
    grid = (E // BR,)
    d1, d2, d3 = w1.shape[1], w2.shape[1], w3.shape[1]

    def body(ef_ref, w1_ref, b1_ref, w2_ref, b2_ref, w3_ref, b3_ref,
             q1_ref, q2_ref, q3_ref):
        x = ef_ref[...]
        q1_ref[...] = jnp.dot(x, w1_ref[...],
                              preferred_element_type=jnp.float32) + b1_ref[...]
        q2_ref[...] = jnp.dot(x, w2_ref[...],
                              preferred_element_type=jnp.float32) + b2_ref[...]
        q3_ref[...] = jnp.dot(x, w3_ref[...],
                              preferred_element_type=jnp.float32) + b3_ref[...]

    wspec = lambda d: pl.BlockSpec((DE, d), lambda i: (0, 0))
    bspec = lambda d: pl.BlockSpec((1, d), lambda i: (0, 0))
    qspec = lambda d: pl.BlockSpec((BR, d), lambda i: (i, 0))
    return pl.pallas_call(
        body,
        grid=grid,
        in_specs=[pl.BlockSpec((BR, DE), lambda i: (i, 0)),
                  wspec(d1), bspec(d1), wspec(d2), bspec(d2),
                  wspec(d3), bspec(d3)],
        out_specs=[qspec(d1), qspec(d2), qspec(d3)],
        out_shape=[jax.ShapeDtypeStruct((E, d1), jnp.float32),
                   jax.ShapeDtypeStruct((E, d2), jnp.float32),
                   jax.ShapeDtypeStruct((E, d3), jnp.float32)],
    )(ef, w1, b1, w2, b2, w3, b3)


def _matmul(a, b):
    def body(a_ref, b_ref, o_ref):
        o_ref[...] = jnp.dot(a_ref[...], b_ref[...],
                             preferred_element_type=jnp.float32)
    return pl.pallas_call(
        body,
        out_shape=jax.ShapeDtypeStruct((a.shape[0], b.shape[1]), jnp.float32),
    )(a, b)


def _node_update(h, parts, was, wan, ba, wmn=None):
    n = h.shape[0]
    dpo = was.shape[1]

    if wmn is None:
        def body(h_ref, pr_ref, was_ref, wan_ref, ba_ref, ho_ref):
            hn = pr_ref[0] + pr_ref[1]
            ho_ref[...] = jnp.maximum(
                jnp.dot(h_ref[...], was_ref[...],
                        preferred_element_type=jnp.float32)
                + jnp.dot(hn, wan_ref[...],
                          preferred_element_type=jnp.float32)
                + ba_ref[...], 0.0)
        return pl.pallas_call(
            body,
            out_shape=jax.ShapeDtypeStruct((n, dpo), jnp.float32),
        )(h, parts, was, wan, ba)

    def body(h_ref, pr_ref, was_ref, wan_ref, ba_ref, wmn_ref,
             ho_ref, po_ref):
        hn = pr_ref[0] + pr_ref[1]
        hnew = jnp.maximum(
            jnp.dot(h_ref[...], was_ref[...],
                    preferred_element_type=jnp.float32)
            + jnp.dot(hn, wan_ref[...], preferred_element_type=jnp.float32)
            + ba_ref[...], 0.0)
        ho_ref[...] = hnew
        po_ref[...] = jnp.dot(hnew, wmn_ref[...],
                              preferred_element_type=jnp.float32)
    return pl.pallas_call(
        body,
        out_shape=[jax.ShapeDtypeStruct((n, dpo), jnp.float32),
                   jax.ShapeDtypeStruct((n, wmn.shape[1]), jnp.float32)],
    )(h, parts, was, wan, ba, wmn)


def _pad2(a, r, c):
    return jnp.pad(a, ((0, r - a.shape[0]), (0, c - a.shape[1])))


def kernel(nfeats, efeats, edge_index, Wm1, bm1, Wa1, ba1,
           Wm2, bm2, Wa2, ba2, Wm3, bm3, Wa3, ba3):
    h0 = jnp.pad(nfeats.reshape(N, DIN), ((0, NP - N), (0, 0)))
    ef = efeats.reshape(E, DE)
    src = edge_index[0]
    dst = edge_index[1]

    dins = (DIN, 50, 25)
    dpis = (128, 64, 32)
    dpos = (64, 32, 128)

    wmn, wme, bmp, was, wan, bap = [], [], [], [], [], []
    for li, (Wm, bm, Wa, ba) in enumerate(
            [(Wm1, bm1, Wa1, ba1), (Wm2, bm2, Wa2, ba2), (Wm3, bm3, Wa3, ba3)]):
        din, dpi, dpo = dins[li], dpis[li], dpos[li]
        wmn.append(_pad2(Wm[:, :din].T, dpi, dpo))
        wme.append(_pad2(Wm[:, din:].T, DE, dpo))
        bmp.append(_pad2(bm[None, :], 1, dpo))
        was.append(_pad2(Wa[:, :din].T, dpi, dpo))
        wan.append(_pad2(Wa[:, din:].T, dpo, dpo))
        bap.append(_pad2(ba[None, :], 1, dpo))

    q1, q2, q3 = _q_prep(ef, wme[0], bmp[0], wme[1], bmp[1], wme[2], bmp[2])

    p1 = _matmul(h0, wmn[0])
    parts1 = _edge_kernel(dpos[0])(p1, q1, src, dst)
    h1, p2 = _node_update(h0, parts1, was[0], wan[0], bap[0], wmn[1])

    parts2 = _edge_kernel(dpos[1])(p2, q2, src, dst)
    h2, p3 = _node_update(h1, parts2, was[1], wan[1], bap[1], wmn[2])

    parts3 = _edge_kernel(dpos[2])(p3, q3, src, dst)
    h3 = _node_update(h2, parts3, was[2], wan[2], bap[2])
    return h3[:N]
